# baseline (speedup 1.0000x reference)
import jax
import jax.numpy as jnp
from jax.experimental import pallas as pl
from jax.experimental.pallas import tpu as pltpu

K_POOL, E = 4, 8
EPS = 1e-6
NJ = 2


def _swiglu_acc(h, w, gw, uw, dw, acc_ref, init):
    g = jnp.dot(h, gw, preferred_element_type=jnp.float32)
    u = jnp.dot(h, uw, preferred_element_type=jnp.float32)
    a = (jax.nn.silu(g) * u) * w
    contrib = jnp.dot(a, dw, preferred_element_type=jnp.float32)

    @pl.when(init)
    def _():
        acc_ref[...] = contrib

    @pl.when(jnp.logical_not(init))
    def _():
        acc_ref[...] += contrib


def _moe_kernel(x_ref, lnpre_ref, router_ref, shg_ref, shu_ref, shd_ref,
                eg_ref, eu_ref, ed_ref, ls_ref, lnpost_ref, out_ref,
                h_ref, acc_ref, w1_ref, w2_ref, i1_ref, i2_ref):
    j = pl.program_id(0)
    e = pl.program_id(1)

    @pl.when((j == 0) & (e == 0))
    def _prologue():
        hf = x_ref[...]
        var = jnp.mean(hf * hf, axis=-1, keepdims=True)
        hf = hf * jax.lax.rsqrt(var + EPS) * lnpre_ref[...]
        h_ref[...] = hf
        logits = jnp.dot(hf, router_ref[...],
                         preferred_element_type=jnp.float32)
        m = jnp.max(logits, axis=-1, keepdims=True)
        p = jnp.exp(logits - m)
        m1 = jnp.max(p, axis=-1, keepdims=True)
        i1 = jnp.argmax(p, axis=-1, keepdims=True)
        lane = jax.lax.broadcasted_iota(jnp.int32, p.shape, 1)
        p2 = jnp.where(lane == i1, -jnp.inf, p)
        m2 = jnp.max(p2, axis=-1, keepdims=True)
        i2 = jnp.argmax(p2, axis=-1, keepdims=True)
        s = m1 + m2
        w1_ref[...] = m1 / s
        w2_ref[...] = m2 / s
        i1_ref[...] = i1.astype(jnp.int32)
        i2_ref[...] = i2.astype(jnp.int32)

    h = h_ref[...]
    init = (j == 0) & (e == 0)

    @pl.when(e == 0)
    def _shared():
        _swiglu_acc(h, 1.0, shg_ref[...], shu_ref[...], shd_ref[...],
                    acc_ref, init)

    @pl.when(e > 0)
    def _routed():
        ex = e - 1
        w_e = (w1_ref[...] * (i1_ref[...] == ex).astype(jnp.float32)
               + w2_ref[...] * (i2_ref[...] == ex).astype(jnp.float32))
        _swiglu_acc(h, w_e, eg_ref[0], eu_ref[0], ed_ref[0], acc_ref, init)

    @pl.when((j == NJ - 1) & (e == E))
    def _epilogue():
        o = acc_ref[...] * ls_ref[...]
        var = jnp.mean(o * o, axis=-1, keepdims=True)
        out_ref[...] = o * jax.lax.rsqrt(var + EPS) * lnpost_ref[...]


@jax.jit
def kernel(x, ln_pre_w, router_w, sh_gate, sh_up, sh_down, eg, eu, ed,
           layer_scale, ln_post_w):
    b, t, d = x.shape
    t2 = (t // K_POOL) * K_POOL
    n = t2 // K_POOL
    in_dim = d * K_POOL
    xs = x[:, :t2, :].reshape(n, in_dim)

    out_dim = sh_down.shape[-1]
    hid = sh_gate.shape[-1]
    n_e = eg.shape[0]
    h2 = hid // NJ

    whole = lambda s: pl.BlockSpec(s, lambda j, e: (0,) * len(s))
    grid_spec = pltpu.PrefetchScalarGridSpec(
        num_scalar_prefetch=0,
        grid=(NJ, n_e + 1),
        in_specs=[
            whole((n, in_dim)),
            whole((1, in_dim)),
            whole((in_dim, n_e)),
            pl.BlockSpec((in_dim, h2), lambda j, e: (0, j)),
            pl.BlockSpec((in_dim, h2), lambda j, e: (0, j)),
            pl.BlockSpec((h2, out_dim), lambda j, e: (j, 0)),
            pl.BlockSpec((1, in_dim, h2),
                         lambda j, e: (jnp.maximum(e - 1, 0), 0, j)),
            pl.BlockSpec((1, in_dim, h2),
                         lambda j, e: (jnp.maximum(e - 1, 0), 0, j)),
            pl.BlockSpec((1, h2, out_dim),
                         lambda j, e: (jnp.maximum(e - 1, 0), j, 0)),
            whole((1, out_dim)),
            whole((1, out_dim)),
        ],
        out_specs=whole((n, out_dim)),
        scratch_shapes=[
            pltpu.VMEM((n, in_dim), jnp.float32),
            pltpu.VMEM((n, out_dim), jnp.float32),
            pltpu.VMEM((n, 1), jnp.float32),
            pltpu.VMEM((n, 1), jnp.float32),
            pltpu.VMEM((n, 1), jnp.int32),
            pltpu.VMEM((n, 1), jnp.int32),
        ],
    )
    out = pl.pallas_call(
        _moe_kernel,
        grid_spec=grid_spec,
        out_shape=jax.ShapeDtypeStruct((n, out_dim), jnp.float32),
        compiler_params=pltpu.CompilerParams(
            dimension_semantics=("arbitrary", "arbitrary"),
        ),
    )(xs, ln_pre_w.reshape(1, in_dim), router_w, sh_gate, sh_up, sh_down,
      eg, eu, ed, layer_scale.reshape(1, out_dim),
      ln_post_w.reshape(1, out_dim))
    return out.reshape(b, n, out_dim)


# speedup vs baseline: 1.2558x; 1.2558x over previous
import jax
import jax.numpy as jnp
from jax.experimental import pallas as pl
from jax.experimental.pallas import tpu as pltpu

K_POOL, E = 4, 8
EPS = 1e-6


def _swiglu_acc(h, w, gw, uw, dw, acc_ref, init):
    g = jnp.dot(h, gw, preferred_element_type=jnp.float32)
    u = jnp.dot(h, uw, preferred_element_type=jnp.float32)
    a = (jax.nn.silu(g) * u) * w
    contrib = jnp.dot(a, dw, preferred_element_type=jnp.float32)

    @pl.when(init)
    def _():
        acc_ref[...] = contrib

    @pl.when(jnp.logical_not(init))
    def _():
        acc_ref[...] += contrib


def _moe_kernel(x_ref, lnpre_ref, router_ref, shg_ref, shu_ref, shd_ref,
                eg_ref, eu_ref, ed_ref, ls_ref, lnpost_ref, out_ref,
                h_ref, acc_ref, w1_ref, w2_ref, i1_ref, i2_ref):
    e = pl.program_id(0)
    n, in_dim = h_ref.shape

    @pl.when(e == 0)
    def _prologue():
        hf = x_ref[...].reshape(n, in_dim)
        var = jnp.mean(hf * hf, axis=-1, keepdims=True)
        hf = hf * jax.lax.rsqrt(var + EPS) * lnpre_ref[...]
        h_ref[...] = hf
        logits = jnp.dot(hf, router_ref[...],
                         preferred_element_type=jnp.float32)
        m = jnp.max(logits, axis=-1, keepdims=True)
        p = jnp.exp(logits - m)
        m1 = jnp.max(p, axis=-1, keepdims=True)
        i1 = jnp.argmax(p, axis=-1, keepdims=True)
        lane = jax.lax.broadcasted_iota(jnp.int32, p.shape, 1)
        p2 = jnp.where(lane == i1, -jnp.inf, p)
        m2 = jnp.max(p2, axis=-1, keepdims=True)
        i2 = jnp.argmax(p2, axis=-1, keepdims=True)
        s = m1 + m2
        w1_ref[...] = m1 / s
        w2_ref[...] = m2 / s
        i1_ref[...] = i1.astype(jnp.int32)
        i2_ref[...] = i2.astype(jnp.int32)
        _swiglu_acc(h_ref[...], 1.0, shg_ref[...], shu_ref[...],
                    shd_ref[...], acc_ref, True)

    @pl.when(e > 0)
    def _routed():
        ex = e - 1
        w_e = (w1_ref[...] * (i1_ref[...] == ex).astype(jnp.float32)
               + w2_ref[...] * (i2_ref[...] == ex).astype(jnp.float32))
        _swiglu_acc(h_ref[...], w_e, eg_ref[0], eu_ref[0], ed_ref[0],
                    acc_ref, False)

    @pl.when(e == E)
    def _epilogue():
        o = acc_ref[...] * ls_ref[...]
        var = jnp.mean(o * o, axis=-1, keepdims=True)
        out_ref[...] = o * jax.lax.rsqrt(var + EPS) * lnpost_ref[...]


@jax.jit
def kernel(x, ln_pre_w, router_w, sh_gate, sh_up, sh_down, eg, eu, ed,
           layer_scale, ln_post_w):
    b, t, d = x.shape
    t2 = (t // K_POOL) * K_POOL
    n = t2 // K_POOL
    in_dim = d * K_POOL
    xs = x.reshape(t, d)[:t2]

    out_dim = sh_down.shape[-1]
    hid = sh_gate.shape[-1]
    n_e = eg.shape[0]

    whole = lambda s: pl.BlockSpec(s, lambda e: (0,) * len(s))
    grid_spec = pltpu.PrefetchScalarGridSpec(
        num_scalar_prefetch=0,
        grid=(n_e + 1,),
        in_specs=[
            whole((t2, d)),
            whole((1, in_dim)),
            whole((in_dim, n_e)),
            whole((in_dim, hid)),
            whole((in_dim, hid)),
            whole((hid, out_dim)),
            pl.BlockSpec((1, in_dim, hid),
                         lambda e: (jnp.maximum(e - 1, 0), 0, 0)),
            pl.BlockSpec((1, in_dim, hid),
                         lambda e: (jnp.maximum(e - 1, 0), 0, 0)),
            pl.BlockSpec((1, hid, out_dim),
                         lambda e: (jnp.maximum(e - 1, 0), 0, 0)),
            whole((1, out_dim)),
            whole((1, out_dim)),
        ],
        out_specs=whole((n, out_dim)),
        scratch_shapes=[
            pltpu.VMEM((n, in_dim), jnp.float32),
            pltpu.VMEM((n, out_dim), jnp.float32),
            pltpu.VMEM((n, 1), jnp.float32),
            pltpu.VMEM((n, 1), jnp.float32),
            pltpu.VMEM((n, 1), jnp.int32),
            pltpu.VMEM((n, 1), jnp.int32),
        ],
    )
    out = pl.pallas_call(
        _moe_kernel,
        grid_spec=grid_spec,
        out_shape=jax.ShapeDtypeStruct((n, out_dim), jnp.float32),
        compiler_params=pltpu.CompilerParams(
            dimension_semantics=("arbitrary",),
        ),
    )(xs, ln_pre_w.reshape(1, in_dim), router_w, sh_gate, sh_up, sh_down,
      eg, eu, ed, layer_scale.reshape(1, out_dim),
      ln_post_w.reshape(1, out_dim))
    return out.reshape(b, n, out_dim)
